# Initial kernel scaffold; baseline (speedup 1.0000x reference)
import jax
import jax.numpy as jnp
from jax.experimental import pallas as pl


def kernel(inputs, unnormalized_widths, unnormalized_heights, unnormalized_derivatives):
    raise NotImplementedError("write your pallas kernel here")


# speedup vs baseline: 382.7092x; 382.7092x over previous
import jax
import jax.numpy as jnp
from jax import lax
from jax.experimental import pallas as pl
from jax.experimental.pallas import tpu as pltpu
from jax.experimental.pallas import tpu_sc as plsc
import numpy as np

BATCH = 8192
VARS = 256
NB = 30
NC, NS, L = 2, 16, 16
NW = NC * NS
ROWS_PER_TILE = BATCH // NW
CH = 64
NG = VARS // L

MBW = 0.001
MIND = 0.001
DCONST = float(np.log(np.exp(1.0 - MIND) - 1.0))

CW_O = 0
W_O = 32
CHT_O = 64
H_O = 96
DL_O = 128
D_O = 160
D1_O = 192
TROWS = 224

LN2 = 0.6931471805599453
SQRT2 = 1.4142135623730951


def _plog(v):
    u = lax.bitcast_convert_type(v, jnp.int32)
    e = lax.shift_right_logical(u, 23) - 127
    m = lax.bitcast_convert_type((u & 0x007FFFFF) | 0x3F800000, jnp.float32)
    big = m > SQRT2
    m = jnp.where(big, m * 0.5, m)
    e = (e + big.astype(jnp.int32)).astype(jnp.float32)
    s = (m - 1.0) / (m + 1.0)
    z = s * s
    p = 2.0 * s * (1.0 + z * (1.0 / 3.0 + z * (0.2 + z * (1.0 / 7.0 + z * (1.0 / 9.0)))))
    return e * LN2 + p


def _softplus(x):
    return _plog(1.0 + jnp.exp(x))


def _sc_body(x_hbm, uw_hbm, uh_hbm, ud_hbm, out_hbm, lad_hbm,
             stg, stgd, tabs, inbuf, obuf, lbuf, sem):
    c = lax.axis_index("c")
    s = lax.axis_index("s")
    wid = s * NC + c
    iota = lax.iota(jnp.int32, L)

    def build(cum_o, val_o):
        def group(g, carry):
            cb = g * L

            def col(k):
                return stg[k, pl.ds(cb, L)]

            m = col(0)
            for k in range(1, NB):
                m = jnp.maximum(m, col(k))
            tot = jnp.zeros((L,), jnp.float32)
            for k in range(NB):
                e = jnp.exp(col(k) - m)
                tabs[pl.ds((val_o + k) * VARS + cb, L)] = e
                tot = tot + e
            rs = (1.0 - MBW * NB) / tot
            acc = jnp.zeros((L,), jnp.float32)
            prev = jnp.full((L,), -5.0, jnp.float32)
            tabs[pl.ds(cum_o * VARS + cb, L)] = prev
            for k in range(NB):
                acc = acc + (MBW + tabs[pl.ds((val_o + k) * VARS + cb, L)] * rs)
                cur = (jnp.full((L,), 5.0, jnp.float32) if k == NB - 1
                       else 10.0 * acc - 5.0)
                tabs[pl.ds((cum_o + k + 1) * VARS + cb, L)] = cur
                tabs[pl.ds((val_o + k) * VARS + cb, L)] = cur - prev
                prev = cur
            tabs[pl.ds((cum_o + NB + 1) * VARS + cb, L)] = jnp.full(
                (L,), 5.0, jnp.float32)
            return carry

        lax.fori_loop(0, NG, group, 0)

    pltpu.sync_copy(uw_hbm, stg)
    build(CW_O, W_O)
    pltpu.sync_copy(uh_hbm, stg)
    build(CHT_O, H_O)
    pltpu.sync_copy(ud_hbm, stgd)

    def group2(g, carry):
        cb = g * L
        for k in range(NB):
            tabs[pl.ds((DL_O + k) * VARS + cb, L)] = (
                tabs[pl.ds((H_O + k) * VARS + cb, L)]
                / tabs[pl.ds((W_O + k) * VARS + cb, L)])
        dconst = jnp.full((L,), DCONST, jnp.float32)
        for k in range(NB + 1):
            if k == 0 or k == NB:
                raw = dconst
            else:
                raw = stgd[k - 1, pl.ds(cb, L)]
            d = MIND + _softplus(raw)
            tabs[pl.ds((D_O + k) * VARS + cb, L)] = d
            if k >= 1:
                tabs[pl.ds((D1_O + k - 1) * VARS + cb, L)] = d
        return carry

    lax.fori_loop(0, NG, group2, 0)

    def process(v, carry):
        r = lax.shift_right_logical(v, 4)
        cb = lax.shift_left(v & (NG - 1), 4)
        lanes = cb + iota
        x0 = inbuf[r, pl.ds(cb, L)]
        x = jnp.clip(x0, -5.0, 5.0)
        b = jnp.zeros((L,), jnp.int32)
        for step in (16, 8, 4, 2, 1):
            cand = b + step
            cv = plsc.load_gather(tabs, [(CW_O + cand) * VARS + lanes])
            ok = (cand <= NB - 1) & (cv <= x)
            b = jnp.where(ok, cand, b)
        base = b * VARS + lanes
        g = lambda off: plsc.load_gather(tabs, [base + (off * VARS)])
        cw_b = g(CW_O)
        w_b = g(W_O)
        ch_b = g(CHT_O)
        h_b = g(H_O)
        dl_b = g(DL_O)
        d_b = g(D_O)
        d1_b = g(D1_O)
        theta = (x - cw_b) / w_b
        omt = 1.0 - theta
        tomt = theta * omt
        th2 = theta * theta
        num = h_b * (dl_b * th2 + d_b * tomt)
        den = dl_b + (d_b + d1_b - 2.0 * dl_b) * tomt
        out_sp = ch_b + num / den
        dn = dl_b * dl_b * (d1_b * th2 + 2.0 * dl_b * tomt + d_b * (omt * omt))
        lad_sp = _plog(dn / (den * den))
        inside = (x0 >= -5.0) & (x0 <= 5.0)
        obuf[r, pl.ds(cb, L)] = jnp.where(inside, out_sp, x0)
        lbuf[r, pl.ds(cb, L)] = jnp.where(inside, lad_sp, 0.0)
        return carry

    for chunk in range(ROWS_PER_TILE // CH):
        r0 = wid * ROWS_PER_TILE + chunk * CH
        pltpu.sync_copy(x_hbm.at[pl.ds(r0, CH), :], inbuf)
        lax.fori_loop(0, CH * NG, process, 0)
        pltpu.sync_copy(obuf, out_hbm.at[pl.ds(r0, CH), :])
        pltpu.sync_copy(lbuf, lad_hbm.at[pl.ds(r0, CH), :])


@jax.jit
def _run(inputs, uw, uh, ud):
    mesh = plsc.VectorSubcoreMesh(
        core_axis_name="c", subcore_axis_name="s", num_cores=NC, num_subcores=NS
    )
    f = pl.kernel(
        _sc_body,
        out_type=(
            jax.ShapeDtypeStruct((BATCH, VARS), jnp.float32),
            jax.ShapeDtypeStruct((BATCH, VARS), jnp.float32),
        ),
        mesh=mesh,
        scratch_types=[
            pltpu.VMEM((NB, VARS), jnp.float32),
            pltpu.VMEM((NB - 1, VARS), jnp.float32),
            pltpu.VMEM((TROWS * VARS,), jnp.float32),
            pltpu.VMEM((CH, VARS), jnp.float32),
            pltpu.VMEM((CH, VARS), jnp.float32),
            pltpu.VMEM((CH, VARS), jnp.float32),
            pltpu.SemaphoreType.DMA,
        ],
        name="rq_spline_sc",
        compiler_params=pltpu.CompilerParams(needs_layout_passes=False),
    )
    return f(inputs, uw.T, uh.T, ud.T)


def kernel(inputs, unnormalized_widths, unnormalized_heights, unnormalized_derivatives):
    return _run(inputs, unnormalized_widths, unnormalized_heights,
                unnormalized_derivatives)


# speedup vs baseline: 1017.8246x; 2.6595x over previous
import jax
import jax.numpy as jnp
from jax import lax
from jax.experimental import pallas as pl
from jax.experimental.pallas import tpu as pltpu
from jax.experimental.pallas import tpu_sc as plsc
import numpy as np

BATCH = 8192
VARS = 256
NB = 30
NC, NS, L = 2, 16, 16
NW = NC * NS
ROWS_PER_TILE = BATCH // NW
CH = 64
NG = VARS // L

MBW = 0.001
MIND = 0.001
DCONST = float(np.log(np.exp(1.0 - MIND) - 1.0))

CW_O = 0
W_O = 32
CHT_O = 64
H_O = 96
DL_O = 128
D_O = 160
D1_O = 192
TROWS = 224

LN2 = 0.6931471805599453
SQRT2 = 1.4142135623730951


def _plog(v):
    u = lax.bitcast_convert_type(v, jnp.int32)
    e = lax.shift_right_logical(u, 23) - 127
    m = lax.bitcast_convert_type((u & 0x007FFFFF) | 0x3F800000, jnp.float32)
    big = m > SQRT2
    m = jnp.where(big, m * 0.5, m)
    e = (e + big.astype(jnp.int32)).astype(jnp.float32)
    s = (m - 1.0) / (m + 1.0)
    z = s * s
    p = 2.0 * s * (1.0 + z * (1.0 / 3.0 + z * (0.2 + z * (1.0 / 7.0 + z * (1.0 / 9.0)))))
    return e * LN2 + p


def _softplus(x):
    return _plog(1.0 + jnp.exp(x))


def _sc_body(x_hbm, uw_hbm, uh_hbm, ud_hbm, out_hbm, lad_hbm,
             stg, stgd, tabs, inbuf, obuf, lbuf, sem):
    c = lax.axis_index("c")
    s = lax.axis_index("s")
    wid = s * NC + c
    iota = lax.iota(jnp.int32, L)

    def build(cum_o, val_o):
        @plsc.parallel_loop(0, NG, unroll=2)
        def group(g):
            cb = g * L

            def col(k):
                return stg[k, pl.ds(cb, L)]

            m = col(0)
            for k in range(1, NB):
                m = jnp.maximum(m, col(k))
            tot = jnp.zeros((L,), jnp.float32)
            for k in range(NB):
                e = jnp.exp(col(k) - m)
                tabs[pl.ds((val_o + k) * VARS + cb, L)] = e
                tot = tot + e
            rs = (1.0 - MBW * NB) / tot
            acc = jnp.zeros((L,), jnp.float32)
            prev = jnp.full((L,), -5.0, jnp.float32)
            tabs[pl.ds(cum_o * VARS + cb, L)] = prev
            for k in range(NB):
                acc = acc + (MBW + tabs[pl.ds((val_o + k) * VARS + cb, L)] * rs)
                cur = (jnp.full((L,), 5.0, jnp.float32) if k == NB - 1
                       else 10.0 * acc - 5.0)
                tabs[pl.ds((cum_o + k + 1) * VARS + cb, L)] = cur
                tabs[pl.ds((val_o + k) * VARS + cb, L)] = cur - prev
                prev = cur
            tabs[pl.ds((cum_o + NB + 1) * VARS + cb, L)] = jnp.full(
                (L,), 5.0, jnp.float32)

    pltpu.sync_copy(uw_hbm, stg)
    build(CW_O, W_O)
    pltpu.sync_copy(uh_hbm, stg)
    build(CHT_O, H_O)
    pltpu.sync_copy(ud_hbm, stgd)

    @plsc.parallel_loop(0, NG, unroll=2)
    def group2(g):
        cb = g * L
        for k in range(NB):
            tabs[pl.ds((DL_O + k) * VARS + cb, L)] = (
                tabs[pl.ds((H_O + k) * VARS + cb, L)]
                / tabs[pl.ds((W_O + k) * VARS + cb, L)])
        dconst = jnp.full((L,), DCONST, jnp.float32)
        for k in range(NB + 1):
            if k == 0 or k == NB:
                raw = dconst
            else:
                raw = stgd[k - 1, pl.ds(cb, L)]
            d = MIND + _softplus(raw)
            tabs[pl.ds((D_O + k) * VARS + cb, L)] = d
            if k >= 1:
                tabs[pl.ds((D1_O + k - 1) * VARS + cb, L)] = d

    def process(v):
        r = lax.shift_right_logical(v, 4)
        cb = lax.shift_left(v & (NG - 1), 4)
        lanes = cb + iota
        x0 = inbuf[r, pl.ds(cb, L)]
        x = jnp.clip(x0, -5.0, 5.0)
        b = jnp.zeros((L,), jnp.int32)
        for step in (16, 8, 4, 2, 1):
            cand = b + step
            cv = plsc.load_gather(tabs, [(CW_O + cand) * VARS + lanes])
            ok = (cand <= NB - 1) & (cv <= x)
            b = jnp.where(ok, cand, b)
        base = b * VARS + lanes
        g = lambda off: plsc.load_gather(tabs, [base + (off * VARS)])
        cw_b = g(CW_O)
        w_b = g(W_O)
        ch_b = g(CHT_O)
        dl_b = g(DL_O)
        d_b = g(D_O)
        d1_b = g(D1_O)
        h_b = dl_b * w_b
        theta = (x - cw_b) / w_b
        omt = 1.0 - theta
        tomt = theta * omt
        th2 = theta * theta
        num = h_b * (dl_b * th2 + d_b * tomt)
        den = dl_b + (d_b + d1_b - 2.0 * dl_b) * tomt
        out_sp = ch_b + num / den
        dn = dl_b * dl_b * (d1_b * th2 + 2.0 * dl_b * tomt + d_b * (omt * omt))
        lad_sp = _plog(dn / (den * den))
        inside = (x0 >= -5.0) & (x0 <= 5.0)
        obuf[r, pl.ds(cb, L)] = jnp.where(inside, out_sp, x0)
        lbuf[r, pl.ds(cb, L)] = jnp.where(inside, lad_sp, 0.0)

    for chunk in range(ROWS_PER_TILE // CH):
        r0 = wid * ROWS_PER_TILE + chunk * CH
        pltpu.sync_copy(x_hbm.at[pl.ds(r0, CH), :], inbuf)
        plsc.parallel_loop(0, CH * NG, unroll=4)(process)
        pltpu.sync_copy(obuf, out_hbm.at[pl.ds(r0, CH), :])
        pltpu.sync_copy(lbuf, lad_hbm.at[pl.ds(r0, CH), :])


@jax.jit
def _run(inputs, uw, uh, ud):
    mesh = plsc.VectorSubcoreMesh(
        core_axis_name="c", subcore_axis_name="s", num_cores=NC, num_subcores=NS
    )
    f = pl.kernel(
        _sc_body,
        out_type=(
            jax.ShapeDtypeStruct((BATCH, VARS), jnp.float32),
            jax.ShapeDtypeStruct((BATCH, VARS), jnp.float32),
        ),
        mesh=mesh,
        scratch_types=[
            pltpu.VMEM((NB, VARS), jnp.float32),
            pltpu.VMEM((NB - 1, VARS), jnp.float32),
            pltpu.VMEM((TROWS * VARS,), jnp.float32),
            pltpu.VMEM((CH, VARS), jnp.float32),
            pltpu.VMEM((CH, VARS), jnp.float32),
            pltpu.VMEM((CH, VARS), jnp.float32),
            pltpu.SemaphoreType.DMA,
        ],
        name="rq_spline_sc",
        compiler_params=pltpu.CompilerParams(needs_layout_passes=False),
    )
    return f(inputs, uw.T, uh.T, ud.T)


def kernel(inputs, unnormalized_widths, unnormalized_heights, unnormalized_derivatives):
    return _run(inputs, unnormalized_widths, unnormalized_heights,
                unnormalized_derivatives)
